# baseline (device time: 179632 ns/iter reference)
import jax
import jax.numpy as jnp
from jax import lax
from jax.experimental import pallas as pl
from jax.experimental.pallas import tpu as pltpu

TILE_WIDTHS = (128, 128, 256, 256, 384, 384)
T = len(TILE_WIDTHS)
TILE_OFFS = tuple(sum(TILE_WIDTHS[:i]) for i in range(T))


def kernel(A, B):
    A = A.astype(jnp.bfloat16)
    m, k = A.shape
    n = B.shape[1]
    n_half = n // 2
    assert sum(TILE_WIDTHS) == n_half

    my_y_out = lax.axis_index("y")
    B_half = lax.dynamic_slice_in_dim(
        B, my_y_out * n_half, n_half, axis=1
    ).astype(jnp.bfloat16)

    def body(a_ref, b_ref, out_ref, recv_ref,
             send_sems_x, recv_sems_x, send_sems_y, recv_sems_y):
        my_x = lax.axis_index("x")
        my_y = lax.axis_index("y")
        x_nbr = (1 - my_x, my_y)
        y_nbr = (my_x, 1 - my_y)

        def out_tile(t):
            return pl.ds(my_y * n_half + TILE_OFFS[t], TILE_WIDTHS[t])

        def half_tile(t):
            return pl.ds(TILE_OFFS[t], TILE_WIDTHS[t])

        barrier_sem = pltpu.get_barrier_semaphore()
        for nbr in (x_nbr, y_nbr):
            pl.semaphore_signal(
                barrier_sem, inc=1, device_id=nbr,
                device_id_type=pl.DeviceIdType.MESH,
            )
        pl.semaphore_wait(barrier_sem, 2)

        rdma_x = []
        for t in range(T):
            out_ref[:, out_tile(t)] = jnp.dot(
                a_ref[...],
                b_ref[:, TILE_OFFS[t]:TILE_OFFS[t] + TILE_WIDTHS[t]],
                preferred_element_type=jnp.float32,
            ).astype(jnp.bfloat16)
            r = pltpu.make_async_remote_copy(
                src_ref=out_ref.at[:, out_tile(t)],
                dst_ref=recv_ref.at[:, half_tile(t)],
                send_sem=send_sems_x.at[t],
                recv_sem=recv_sems_x.at[t],
                device_id=x_nbr,
                device_id_type=pl.DeviceIdType.MESH,
            )
            r.start()
            rdma_x.append(r)

        rdma_y = []
        for t in range(T):
            rdma_x[t].wait()
            out_ref[:, out_tile(t)] = (
                out_ref[:, out_tile(t)] + recv_ref[:, half_tile(t)]
            )
            r = pltpu.make_async_remote_copy(
                src_ref=out_ref.at[:, out_tile(t)],
                dst_ref=out_ref.at[:, out_tile(t)],
                send_sem=send_sems_y.at[t],
                recv_sem=recv_sems_y.at[t],
                device_id=y_nbr,
                device_id_type=pl.DeviceIdType.MESH,
            )
            r.start()
            rdma_y.append(r)

        for t in range(T):
            rdma_y[t].wait()

    return pl.pallas_call(
        body,
        out_shape=jax.ShapeDtypeStruct((m, n), jnp.bfloat16),
        in_specs=[
            pl.BlockSpec(memory_space=pltpu.VMEM),
            pl.BlockSpec(memory_space=pltpu.VMEM),
        ],
        out_specs=pl.BlockSpec(memory_space=pltpu.VMEM),
        scratch_shapes=[
            pltpu.VMEM((m, n_half), jnp.bfloat16),
            pltpu.SemaphoreType.DMA((T,)),
            pltpu.SemaphoreType.DMA((T,)),
            pltpu.SemaphoreType.DMA((T,)),
            pltpu.SemaphoreType.DMA((T,)),
        ],
        compiler_params=pltpu.CompilerParams(
            collective_id=0, vmem_limit_bytes=100 * 1024 * 1024
        ),
    )(A, B_half)
